# baseline (speedup 1.0000x reference)
import functools

import jax
import jax.numpy as jnp
from jax import lax
from jax.experimental import pallas as pl
from jax.experimental.pallas import tpu as pltpu
from jax.experimental.pallas import tpu_sc as plsc

N = 10000
E = 320000
D = 128
B = 64
G = 16

NC = 2
NS = 16
NW = NC * NS
NP = 10240
PAD_ROWS = NP - N
CH = 128
NCH = 80
CHA = 80
NCHA = 128
PHA = 16
EP = NW * NCH * CH
RPW = NP // NS

BR = 1024
NBLK = NP // BR


def _zero_rows(ref, nrows, ncols):
  z = jnp.zeros((16,), jnp.float32)

  def body(i, carry):
    for j in range(ncols // 16):
      ref[i, pl.ds(j * 16, 16)] = z
    return carry

  lax.fori_loop(0, nrows, body, 0)


def _fill_ones(ref, nrows, ncols):
  o = jnp.ones((16,), jnp.float32)

  def body(i, carry):
    for j in range(ncols // 16):
      ref[i, pl.ds(j * 16, 16)] = o
    return carry

  lax.fori_loop(0, nrows, body, 0)


@functools.cache
def _make_agg():
  scratch = [
      pltpu.VMEM((PHA, CHA), jnp.int32),
      pltpu.VMEM((PHA, CHA), jnp.int32),
      [pltpu.VMEM((CHA, D), jnp.float32)] * 4,
      pltpu.VMEM_SHARED((NP, D), jnp.float32),
      [pltpu.SemaphoreType.DMA] * 4,
      [pltpu.SemaphoreType.DMA] * 4,
  ]

  def body(x_hbm, src_hbm, dst_hbm, out_hbm, src_v, dst_v, bufs, acc,
           gsems, ssems):
    c = lax.axis_index("c")
    s = lax.axis_index("s")
    w = s * NC + c

    _zero_rows(bufs[0], CHA, D)
    for r in range(RPW // CHA):
      pltpu.sync_copy(bufs[0], acc.at[pl.ds(s * RPW + r * CHA, CHA)])
    plsc.subcore_barrier()

    for p in range(NCHA // PHA):
      pltpu.sync_copy(src_hbm.at[w, pl.ds(p * PHA, PHA)], src_v)
      pltpu.sync_copy(dst_hbm.at[w, pl.ds(p * PHA, PHA)], dst_v)
      for k in range(4):
        pltpu.async_copy(x_hbm.at[src_v.at[k]], bufs[k], gsems[k])

      def wave(j4, carry):
        j = j4 * 4
        for k in range(4):
          pltpu.make_async_copy(x_hbm.at[src_v.at[j + k]], bufs[k],
                                gsems[k]).wait()
          pltpu.async_copy(bufs[k], acc.at[dst_v.at[j + k]], ssems[k],
                           add=True)
        for k in range(4):
          pltpu.make_async_copy(bufs[k], acc.at[dst_v.at[j + k]],
                                ssems[k]).wait()

          @pl.when(j4 < PHA // 4 - 1)
          def _():
            pltpu.async_copy(x_hbm.at[src_v.at[j + 4 + k]], bufs[k],
                             gsems[k])

        return carry

      lax.fori_loop(0, PHA // 4, wave, 0)

    plsc.subcore_barrier()
    pltpu.sync_copy(acc.at[pl.ds(s * RPW, RPW)],
                    out_hbm.at[c, pl.ds(s * RPW, RPW)])

  mesh = plsc.VectorSubcoreMesh(
      core_axis_name="c", subcore_axis_name="s", num_cores=NC, num_subcores=NS)
  return pl.kernel(body,
                   out_type=jax.ShapeDtypeStruct((NC, NP, D), jnp.float32),
                   mesh=mesh, scratch_types=scratch)


@functools.cache
def _make_deg():
  scratch = [
      pltpu.VMEM((NCH, CH), jnp.int32),
      pltpu.VMEM((CH, 16), jnp.float32),
      pltpu.VMEM((CH, 16), jnp.float32),
      pltpu.VMEM_SHARED((NP, 16), jnp.float32),
      pltpu.SemaphoreType.DMA,
  ]

  def body(dst_hbm, out_hbm, dst_v, ones_v, z_v, dacc, dsem):
    c = lax.axis_index("c")
    s = lax.axis_index("s")
    w = s * NC + c

    pltpu.sync_copy(dst_hbm.at[w], dst_v)
    _fill_ones(ones_v, CH, 16)
    _zero_rows(z_v, CH, 16)
    for r in range(RPW // CH):
      pltpu.sync_copy(z_v, dacc.at[pl.ds(s * RPW + r * CH, CH)])
    plsc.subcore_barrier()

    def wave(t, carry):
      for k in range(4):
        pltpu.async_copy(ones_v, dacc.at[dst_v.at[t * 4 + k]], dsem, add=True)
      for k in range(4):
        pltpu.make_async_copy(ones_v, dacc.at[dst_v.at[t * 4 + k]],
                              dsem).wait()
      return carry

    lax.fori_loop(0, NCH // 4, wave, 0)

    plsc.subcore_barrier()
    pltpu.sync_copy(dacc.at[pl.ds(s * RPW, RPW)],
                    out_hbm.at[c, pl.ds(s * RPW, RPW)])

  mesh = plsc.VectorSubcoreMesh(
      core_axis_name="c", subcore_axis_name="s", num_cores=NC, num_subcores=NS)
  return pl.kernel(body,
                   out_type=jax.ShapeDtypeStruct((NC, NP, 16), jnp.float32),
                   mesh=mesh, scratch_types=scratch,
                   compiler_params=pltpu.CompilerParams(
                       use_tc_tiling_on_sc=False))


def _bn_scale_shift(st_ref, g_ref, be_ref):
  mu = st_ref[0:1, :] * (1.0 / N)
  ex2 = st_ref[1:2, :] * (1.0 / N)
  var = ex2 - mu * mu
  scale = lax.rsqrt(var + 1e-5) * g_ref[...]
  shift = be_ref[...] - mu * scale
  return scale, shift


def _compute_h(acc_ref, deg_ref, x_ref, wl_ref, wr_ref, bl_ref):
  d = jnp.maximum(deg_ref[0, :, 0:1] + deg_ref[1, :, 0:1], 1.0)
  m = (acc_ref[0] + acc_ref[1]) / d
  return (lax.dot_general(m, wl_ref[...], (((1,), (1,)), ((), ())),
                          preferred_element_type=jnp.float32)
          + lax.dot_general(x_ref[...], wr_ref[...], (((1,), (1,)), ((), ())),
                            preferred_element_type=jnp.float32)
          + bl_ref[...])


def _accum_stats(i, h, st_ref):
  @pl.when(i == 0)
  def _():
    st_ref[...] = jnp.zeros_like(st_ref)

  rows = lax.broadcasted_iota(jnp.int32, (BR, 1), 0) + i * BR
  hm = jnp.where(rows < N, h, 0.0)
  st_ref[0:1, :] += jnp.sum(hm, axis=0, keepdims=True)
  st_ref[1:2, :] += jnp.sum(hm * hm, axis=0, keepdims=True)


def _layer_body(acc_ref, deg_ref, x_ref, wl_ref, wr_ref, bl_ref, g_ref,
                be_ref, o_ref, hbuf, st_ref):
  p = pl.program_id(0)
  i = pl.program_id(1)
  base = pl.multiple_of(i * BR, BR)

  @pl.when(p == 0)
  def _():
    h = _compute_h(acc_ref, deg_ref, x_ref, wl_ref, wr_ref, bl_ref)
    hbuf[pl.ds(base, BR), :] = h
    _accum_stats(i, h, st_ref)

  @pl.when(p == 1)
  def _():
    scale, shift = _bn_scale_shift(st_ref, g_ref, be_ref)
    normed = jnp.maximum(hbuf[pl.ds(base, BR), :] * scale + shift, 0.0)
    rows = lax.broadcasted_iota(jnp.int32, (BR, 1), 0) + i * BR
    o_ref[...] = jnp.where(rows < N, normed, 0.0)


_layer = pl.pallas_call(
    _layer_body,
    grid=(2, NBLK),
    in_specs=[
        pl.BlockSpec((NC, BR, D), lambda p, i: (0, i * (1 - p), 0)),
        pl.BlockSpec((NC, BR, 16), lambda p, i: (0, i * (1 - p), 0)),
        pl.BlockSpec((BR, D), lambda p, i: (i * (1 - p), 0)),
        pl.BlockSpec((D, D), lambda p, i: (0, 0)),
        pl.BlockSpec((D, D), lambda p, i: (0, 0)),
        pl.BlockSpec((1, D), lambda p, i: (0, 0)),
        pl.BlockSpec((1, D), lambda p, i: (0, 0)),
        pl.BlockSpec((1, D), lambda p, i: (0, 0)),
    ],
    out_specs=pl.BlockSpec((BR, D), lambda p, i: (i, 0)),
    out_shape=jax.ShapeDtypeStruct((NP, D), jnp.float32),
    scratch_shapes=[
        pltpu.VMEM((NP, D), jnp.float32),
        pltpu.VMEM((8, D), jnp.float32),
    ],
    compiler_params=pltpu.CompilerParams(
        dimension_semantics=("arbitrary", "arbitrary")),
)


def _tail_body(acc_ref, deg_ref, x_ref, wl_ref, wr_ref, bl_ref, g_ref,
               be_ref, b_ref, gf_ref, w1a_ref, w1b_ref, b1_ref, w2_ref,
               b2_ref, o_ref, hbuf, st_ref, pooled, cnts):
  p = pl.program_id(0)
  i = pl.program_id(1)
  base = pl.multiple_of(i * BR, BR)

  @pl.when(p == 0)
  def _():
    h = _compute_h(acc_ref, deg_ref, x_ref, wl_ref, wr_ref, bl_ref)
    hbuf[pl.ds(base, BR), :] = h
    _accum_stats(i, h, st_ref)

  @pl.when(p == 1)
  def _():
    @pl.when(i == 0)
    def _():
      pooled[...] = jnp.zeros_like(pooled)
      cnts[...] = jnp.zeros_like(cnts)

    scale, shift = _bn_scale_shift(st_ref, g_ref, be_ref)
    h3 = jnp.maximum(hbuf[pl.ds(base, BR), :] * scale + shift, 0.0)
    b = b_ref[0]
    oh = (b == lax.broadcasted_iota(jnp.int32, (B, BR), 0)).astype(jnp.float32)
    pooled[...] += lax.dot_general(oh, h3, (((1,), (0,)), ((), ())),
                                   preferred_element_type=jnp.float32)
    cnts[...] += lax.dot_general(oh, jnp.ones_like(h3), (((1,), (0,)), ((), ())),
                                 preferred_element_type=jnp.float32)

    @pl.when(i == NBLK - 1)
    def _():
      pm = pooled[...] / jnp.maximum(cnts[...], 1.0)
      z = (lax.dot_general(pm, w1a_ref[...], (((1,), (1,)), ((), ())),
                           preferred_element_type=jnp.float32)
           + lax.dot_general(gf_ref[...], w1b_ref[...], (((1,), (1,)), ((), ())),
                             preferred_element_type=jnp.float32)
           + b1_ref[...])
      z = jnp.maximum(z, 0.0)
      o_ref[...] = (jnp.sum(z * w2_ref[...], axis=1, keepdims=True)
                    + b2_ref[0, 0])


_tail = pl.pallas_call(
    _tail_body,
    grid=(2, NBLK),
    in_specs=[
        pl.BlockSpec((NC, BR, D), lambda p, i: (0, i * (1 - p), 0)),
        pl.BlockSpec((NC, BR, 16), lambda p, i: (0, i * (1 - p), 0)),
        pl.BlockSpec((BR, D), lambda p, i: (i * (1 - p), 0)),
        pl.BlockSpec((D, D), lambda p, i: (0, 0)),
        pl.BlockSpec((D, D), lambda p, i: (0, 0)),
        pl.BlockSpec((1, D), lambda p, i: (0, 0)),
        pl.BlockSpec((1, D), lambda p, i: (0, 0)),
        pl.BlockSpec((1, D), lambda p, i: (0, 0)),
        pl.BlockSpec((1, 1, BR), lambda p, i: (i * p, 0, 0)),
        pl.BlockSpec((B, G), lambda p, i: (0, 0)),
        pl.BlockSpec((B, D), lambda p, i: (0, 0)),
        pl.BlockSpec((B, G), lambda p, i: (0, 0)),
        pl.BlockSpec((1, B), lambda p, i: (0, 0)),
        pl.BlockSpec((1, B), lambda p, i: (0, 0)),
        pl.BlockSpec((1, 1), lambda p, i: (0, 0)),
    ],
    out_specs=pl.BlockSpec((B, 1), lambda p, i: (0, 0)),
    out_shape=jax.ShapeDtypeStruct((B, 1), jnp.float32),
    scratch_shapes=[
        pltpu.VMEM((NP, D), jnp.float32),
        pltpu.VMEM((8, D), jnp.float32),
        pltpu.VMEM((B, D), jnp.float32),
        pltpu.VMEM((B, D), jnp.float32),
    ],
    compiler_params=pltpu.CompilerParams(
        dimension_semantics=("arbitrary", "arbitrary")),
)


def _agg(h, src, dst):
  return _make_agg()(h, src, dst)


def _deg(dst):
  return _make_deg()(dst)


def kernel(x, edge_index, batch, global_features,
           Wl0, Wr0, bl0, g0, be0,
           Wl1, Wr1, bl1, g1, be1,
           Wl2, Wr2, bl2, g2, be2,
           W1, b1, W2, b2):
  npad = EP - E
  pad_idx = N + jnp.arange(npad, dtype=jnp.int32) % PAD_ROWS
  src_flat = jnp.concatenate([edge_index[0], pad_idx])
  dst_flat = jnp.concatenate([edge_index[1], pad_idx])
  src = src_flat.reshape(NW, NCHA, CHA)
  dst = dst_flat.reshape(NW, NCHA, CHA)
  dst_deg = dst_flat.reshape(NW, NCH, CH)
  batch_r = jnp.pad(batch, (0, PAD_ROWS),
                    constant_values=B).reshape(NBLK, 1, BR)
  gf = global_features.astype(jnp.float32).reshape(B, G)
  W1a = W1[:, :D]
  W1b = W1[:, D:]
  b1r = b1.reshape(1, B)
  b2r = b2.reshape(1, 1)

  layers = [(Wl0, Wr0, bl0, g0, be0),
            (Wl1, Wr1, bl1, g1, be1),
            (Wl2, Wr2, bl2, g2, be2)]

  h = jnp.pad(x.astype(jnp.float32), ((0, PAD_ROWS), (0, 0)))
  deg = _deg(dst_deg)
  for Wl, Wr, bl, g, be in layers[:2]:
    agg = _agg(h, src, dst)
    h = _layer(agg, deg, h, Wl, Wr, bl.reshape(1, D),
               g.reshape(1, D), be.reshape(1, D))

  Wl, Wr, bl, g, be = layers[2]
  agg = _agg(h, src, dst)
  return _tail(agg, deg, h, Wl, Wr, bl.reshape(1, D),
               g.reshape(1, D), be.reshape(1, D),
               batch_r, gf, W1a, W1b, b1r, W2.reshape(1, B), b2r)


# speedup vs baseline: 1.0615x; 1.0615x over previous
import functools

import jax
import jax.numpy as jnp
from jax import lax
from jax.experimental import pallas as pl
from jax.experimental.pallas import tpu as pltpu
from jax.experimental.pallas import tpu_sc as plsc

N = 10000
E = 320000
D = 128
B = 64
G = 16

NC = 2
NS = 16
NW = NC * NS
NP = 10240
PAD_ROWS = NP - N
CH = 128
NCH = 80
CHA = 64
NCHA = 160
PHA = 40
EP = NW * NCH * CH
RPW = NP // NS

BR = 1024
NBLK = NP // BR


def _zero_rows(ref, nrows, ncols):
  z = jnp.zeros((16,), jnp.float32)

  def body(i, carry):
    for j in range(ncols // 16):
      ref[i, pl.ds(j * 16, 16)] = z
    return carry

  lax.fori_loop(0, nrows, body, 0)


def _fill_ones(ref, nrows, ncols):
  o = jnp.ones((16,), jnp.float32)

  def body(i, carry):
    for j in range(ncols // 16):
      ref[i, pl.ds(j * 16, 16)] = o
    return carry

  lax.fori_loop(0, nrows, body, 0)


@functools.cache
def _make_agg():
  scratch = [
      pltpu.VMEM((PHA, CHA), jnp.int32),
      pltpu.VMEM((PHA, CHA), jnp.int32),
      [pltpu.VMEM((CHA, D), jnp.float32)] * 4,
      pltpu.VMEM_SHARED((NP, D), jnp.float32),
      [pltpu.SemaphoreType.DMA] * 4,
      [pltpu.SemaphoreType.DMA] * 4,
  ]

  def body(x_hbm, src_hbm, dst_hbm, out_hbm, src_v, dst_v, bufs, acc,
           gsems, ssems):
    c = lax.axis_index("c")
    s = lax.axis_index("s")
    w = s * NC + c

    _zero_rows(bufs[0], CHA, D)
    for r in range(RPW // CHA):
      pltpu.sync_copy(bufs[0], acc.at[pl.ds(s * RPW + r * CHA, CHA)])
    plsc.subcore_barrier()

    for p in range(NCHA // PHA):
      pltpu.sync_copy(src_hbm.at[w, pl.ds(p * PHA, PHA)], src_v)
      pltpu.sync_copy(dst_hbm.at[w, pl.ds(p * PHA, PHA)], dst_v)
      for k in range(4):
        pltpu.async_copy(x_hbm.at[src_v.at[k]], bufs[k], gsems[k])

      def wave(j4, carry):
        j = j4 * 4
        for k in range(4):
          pltpu.make_async_copy(x_hbm.at[src_v.at[j + k]], bufs[k],
                                gsems[k]).wait()
          pltpu.async_copy(bufs[k], acc.at[dst_v.at[j + k]], ssems[k],
                           add=True)
        for k in range(4):
          pltpu.make_async_copy(bufs[k], acc.at[dst_v.at[j + k]],
                                ssems[k]).wait()

          @pl.when(j4 < PHA // 4 - 1)
          def _():
            pltpu.async_copy(x_hbm.at[src_v.at[j + 4 + k]], bufs[k],
                             gsems[k])

        return carry

      lax.fori_loop(0, PHA // 4, wave, 0)

    plsc.subcore_barrier()
    pltpu.sync_copy(acc.at[pl.ds(s * RPW, RPW)],
                    out_hbm.at[c, pl.ds(s * RPW, RPW)])

  mesh = plsc.VectorSubcoreMesh(
      core_axis_name="c", subcore_axis_name="s", num_cores=NC, num_subcores=NS)
  return pl.kernel(body,
                   out_type=jax.ShapeDtypeStruct((NC, NP, D), jnp.float32),
                   mesh=mesh, scratch_types=scratch)


@functools.cache
def _make_deg():
  scratch = [
      pltpu.VMEM((NCH, CH), jnp.int32),
      pltpu.VMEM((CH, 16), jnp.float32),
      pltpu.VMEM((CH, 16), jnp.float32),
      pltpu.VMEM_SHARED((NP, 16), jnp.float32),
      pltpu.SemaphoreType.DMA,
  ]

  def body(dst_hbm, out_hbm, dst_v, ones_v, z_v, dacc, dsem):
    c = lax.axis_index("c")
    s = lax.axis_index("s")
    w = s * NC + c

    pltpu.sync_copy(dst_hbm.at[w], dst_v)
    _fill_ones(ones_v, CH, 16)
    _zero_rows(z_v, CH, 16)
    for r in range(RPW // CH):
      pltpu.sync_copy(z_v, dacc.at[pl.ds(s * RPW + r * CH, CH)])
    plsc.subcore_barrier()

    def wave(t, carry):
      for k in range(4):
        pltpu.async_copy(ones_v, dacc.at[dst_v.at[t * 4 + k]], dsem, add=True)
      for k in range(4):
        pltpu.make_async_copy(ones_v, dacc.at[dst_v.at[t * 4 + k]],
                              dsem).wait()
      return carry

    lax.fori_loop(0, NCH // 4, wave, 0)

    plsc.subcore_barrier()
    pltpu.sync_copy(dacc.at[pl.ds(s * RPW, RPW)],
                    out_hbm.at[c, pl.ds(s * RPW, RPW)])

  mesh = plsc.VectorSubcoreMesh(
      core_axis_name="c", subcore_axis_name="s", num_cores=NC, num_subcores=NS)
  return pl.kernel(body,
                   out_type=jax.ShapeDtypeStruct((NC, NP, 16), jnp.float32),
                   mesh=mesh, scratch_types=scratch,
                   compiler_params=pltpu.CompilerParams(
                       use_tc_tiling_on_sc=False))


def _bn_scale_shift(st_ref, g_ref, be_ref):
  mu = st_ref[0:1, :] * (1.0 / N)
  ex2 = st_ref[1:2, :] * (1.0 / N)
  var = ex2 - mu * mu
  scale = lax.rsqrt(var + 1e-5) * g_ref[...]
  shift = be_ref[...] - mu * scale
  return scale, shift


def _compute_h(acc_ref, deg_ref, x_ref, wl_ref, wr_ref, bl_ref):
  d = jnp.maximum(deg_ref[0, :, 0:1] + deg_ref[1, :, 0:1], 1.0)
  m = (acc_ref[0] + acc_ref[1]) / d
  return (lax.dot_general(m, wl_ref[...], (((1,), (1,)), ((), ())),
                          preferred_element_type=jnp.float32)
          + lax.dot_general(x_ref[...], wr_ref[...], (((1,), (1,)), ((), ())),
                            preferred_element_type=jnp.float32)
          + bl_ref[...])


def _accum_stats(i, h, st_ref):
  @pl.when(i == 0)
  def _():
    st_ref[...] = jnp.zeros_like(st_ref)

  rows = lax.broadcasted_iota(jnp.int32, (BR, 1), 0) + i * BR
  hm = jnp.where(rows < N, h, 0.0)
  st_ref[0:1, :] += jnp.sum(hm, axis=0, keepdims=True)
  st_ref[1:2, :] += jnp.sum(hm * hm, axis=0, keepdims=True)


def _layer_body(acc_ref, deg_ref, x_ref, wl_ref, wr_ref, bl_ref, g_ref,
                be_ref, o_ref, hbuf, st_ref):
  p = pl.program_id(0)
  i = pl.program_id(1)
  base = pl.multiple_of(i * BR, BR)

  @pl.when(p == 0)
  def _():
    h = _compute_h(acc_ref, deg_ref, x_ref, wl_ref, wr_ref, bl_ref)
    hbuf[pl.ds(base, BR), :] = h
    _accum_stats(i, h, st_ref)

  @pl.when(p == 1)
  def _():
    scale, shift = _bn_scale_shift(st_ref, g_ref, be_ref)
    normed = jnp.maximum(hbuf[pl.ds(base, BR), :] * scale + shift, 0.0)
    rows = lax.broadcasted_iota(jnp.int32, (BR, 1), 0) + i * BR
    o_ref[...] = jnp.where(rows < N, normed, 0.0)


_layer = pl.pallas_call(
    _layer_body,
    grid=(2, NBLK),
    in_specs=[
        pl.BlockSpec((NC, BR, D), lambda p, i: (0, i * (1 - p), 0)),
        pl.BlockSpec((NC, BR, 16), lambda p, i: (0, i * (1 - p), 0)),
        pl.BlockSpec((BR, D), lambda p, i: (i * (1 - p), 0)),
        pl.BlockSpec((D, D), lambda p, i: (0, 0)),
        pl.BlockSpec((D, D), lambda p, i: (0, 0)),
        pl.BlockSpec((1, D), lambda p, i: (0, 0)),
        pl.BlockSpec((1, D), lambda p, i: (0, 0)),
        pl.BlockSpec((1, D), lambda p, i: (0, 0)),
    ],
    out_specs=pl.BlockSpec((BR, D), lambda p, i: (i, 0)),
    out_shape=jax.ShapeDtypeStruct((NP, D), jnp.float32),
    scratch_shapes=[
        pltpu.VMEM((NP, D), jnp.float32),
        pltpu.VMEM((8, D), jnp.float32),
    ],
    compiler_params=pltpu.CompilerParams(
        dimension_semantics=("arbitrary", "arbitrary")),
)


def _tail_body(acc_ref, deg_ref, x_ref, wl_ref, wr_ref, bl_ref, g_ref,
               be_ref, b_ref, gf_ref, w1a_ref, w1b_ref, b1_ref, w2_ref,
               b2_ref, o_ref, hbuf, st_ref, pooled, cnts):
  p = pl.program_id(0)
  i = pl.program_id(1)
  base = pl.multiple_of(i * BR, BR)

  @pl.when(p == 0)
  def _():
    h = _compute_h(acc_ref, deg_ref, x_ref, wl_ref, wr_ref, bl_ref)
    hbuf[pl.ds(base, BR), :] = h
    _accum_stats(i, h, st_ref)

  @pl.when(p == 1)
  def _():
    @pl.when(i == 0)
    def _():
      pooled[...] = jnp.zeros_like(pooled)
      cnts[...] = jnp.zeros_like(cnts)

    scale, shift = _bn_scale_shift(st_ref, g_ref, be_ref)
    h3 = jnp.maximum(hbuf[pl.ds(base, BR), :] * scale + shift, 0.0)
    b = b_ref[0]
    oh = (b == lax.broadcasted_iota(jnp.int32, (B, BR), 0)).astype(jnp.float32)
    pooled[...] += lax.dot_general(oh, h3, (((1,), (0,)), ((), ())),
                                   preferred_element_type=jnp.float32)
    cnts[...] += lax.dot_general(oh, jnp.ones_like(h3), (((1,), (0,)), ((), ())),
                                 preferred_element_type=jnp.float32)

    @pl.when(i == NBLK - 1)
    def _():
      pm = pooled[...] / jnp.maximum(cnts[...], 1.0)
      z = (lax.dot_general(pm, w1a_ref[...], (((1,), (1,)), ((), ())),
                           preferred_element_type=jnp.float32)
           + lax.dot_general(gf_ref[...], w1b_ref[...], (((1,), (1,)), ((), ())),
                             preferred_element_type=jnp.float32)
           + b1_ref[...])
      z = jnp.maximum(z, 0.0)
      o_ref[...] = (jnp.sum(z * w2_ref[...], axis=1, keepdims=True)
                    + b2_ref[0, 0])


_tail = pl.pallas_call(
    _tail_body,
    grid=(2, NBLK),
    in_specs=[
        pl.BlockSpec((NC, BR, D), lambda p, i: (0, i * (1 - p), 0)),
        pl.BlockSpec((NC, BR, 16), lambda p, i: (0, i * (1 - p), 0)),
        pl.BlockSpec((BR, D), lambda p, i: (i * (1 - p), 0)),
        pl.BlockSpec((D, D), lambda p, i: (0, 0)),
        pl.BlockSpec((D, D), lambda p, i: (0, 0)),
        pl.BlockSpec((1, D), lambda p, i: (0, 0)),
        pl.BlockSpec((1, D), lambda p, i: (0, 0)),
        pl.BlockSpec((1, D), lambda p, i: (0, 0)),
        pl.BlockSpec((1, 1, BR), lambda p, i: (i * p, 0, 0)),
        pl.BlockSpec((B, G), lambda p, i: (0, 0)),
        pl.BlockSpec((B, D), lambda p, i: (0, 0)),
        pl.BlockSpec((B, G), lambda p, i: (0, 0)),
        pl.BlockSpec((1, B), lambda p, i: (0, 0)),
        pl.BlockSpec((1, B), lambda p, i: (0, 0)),
        pl.BlockSpec((1, 1), lambda p, i: (0, 0)),
    ],
    out_specs=pl.BlockSpec((B, 1), lambda p, i: (0, 0)),
    out_shape=jax.ShapeDtypeStruct((B, 1), jnp.float32),
    scratch_shapes=[
        pltpu.VMEM((NP, D), jnp.float32),
        pltpu.VMEM((8, D), jnp.float32),
        pltpu.VMEM((B, D), jnp.float32),
        pltpu.VMEM((B, D), jnp.float32),
    ],
    compiler_params=pltpu.CompilerParams(
        dimension_semantics=("arbitrary", "arbitrary")),
)


def _agg(h, src, dst):
  return _make_agg()(h, src, dst)


def _deg(dst):
  return _make_deg()(dst)


def kernel(x, edge_index, batch, global_features,
           Wl0, Wr0, bl0, g0, be0,
           Wl1, Wr1, bl1, g1, be1,
           Wl2, Wr2, bl2, g2, be2,
           W1, b1, W2, b2):
  npad = EP - E
  pad_idx = N + jnp.arange(npad, dtype=jnp.int32) % PAD_ROWS
  src_flat = jnp.concatenate([edge_index[0], pad_idx])
  dst_flat = jnp.concatenate([edge_index[1], pad_idx])
  src = src_flat.reshape(NW, NCHA, CHA)
  dst = dst_flat.reshape(NW, NCHA, CHA)
  dst_deg = dst_flat.reshape(NW, NCH, CH)
  batch_r = jnp.pad(batch, (0, PAD_ROWS),
                    constant_values=B).reshape(NBLK, 1, BR)
  gf = global_features.astype(jnp.float32).reshape(B, G)
  W1a = W1[:, :D]
  W1b = W1[:, D:]
  b1r = b1.reshape(1, B)
  b2r = b2.reshape(1, 1)

  layers = [(Wl0, Wr0, bl0, g0, be0),
            (Wl1, Wr1, bl1, g1, be1),
            (Wl2, Wr2, bl2, g2, be2)]

  h = jnp.pad(x.astype(jnp.float32), ((0, PAD_ROWS), (0, 0)))
  deg = _deg(dst_deg)
  for Wl, Wr, bl, g, be in layers[:2]:
    agg = _agg(h, src, dst)
    h = _layer(agg, deg, h, Wl, Wr, bl.reshape(1, D),
               g.reshape(1, D), be.reshape(1, D))

  Wl, Wr, bl, g, be = layers[2]
  agg = _agg(h, src, dst)
  return _tail(agg, deg, h, Wl, Wr, bl.reshape(1, D),
               g.reshape(1, D), be.reshape(1, D),
               batch_r, gf, W1a, W1b, b1r, W2.reshape(1, B), b2r)
